# baseline (speedup 1.0000x reference)
import functools

import jax
import jax.numpy as jnp
from jax import lax
from jax.experimental import pallas as pl
from jax.experimental.pallas import tpu as pltpu
from jax.experimental.pallas import tpu_sc as plsc

NN = 50000
EE = 800000
GG = 512
HH = 64
HALF = 32
NBLK = 2000
NGRID = NN // NBLK
NTILE = 16
ECH = 128
NCH = 392
EP = NTILE * NCH * ECH
NP = 50048
RPT = NP // NTILE
ZR = 136
NBUF = 4


@functools.cache
def _make_sc_agg():
    mesh = plsc.VectorSubcoreMesh(
        core_axis_name="c", subcore_axis_name="s", num_cores=2, num_subcores=16)
    return functools.partial(
        pl.kernel,
        out_type=(
            jax.ShapeDtypeStruct((NP, HALF), jnp.float32),
            jax.ShapeDtypeStruct((NP, HALF), jnp.float32),
        ),
        mesh=mesh,
        scratch_types=[
            pltpu.VMEM((4 * NBUF, ECH), jnp.int32),
            pltpu.VMEM((2 * NBUF * 16,), jnp.int32),
            pltpu.VMEM((NBUF, ECH, HALF), jnp.float32),
            pltpu.VMEM((ZR, HALF), jnp.float32),
            pltpu.VMEM_SHARED((NP, HALF), jnp.float32),
            pltpu.SemaphoreType.DMA,
            pltpu.SemaphoreType.DMA,
            pltpu.SemaphoreType.DMA,
        ],
        compiler_params=pltpu.CompilerParams(use_tc_tiling_on_sc=False),
    )(_sc_agg_body)


def _sc_agg_body(y0, y1, edges, out0, out1,
                 sd_v, rid_v, row_v, z_v, acc, sem_i, sem_g, sem_s):
    c = lax.axis_index("c")
    s = lax.axis_index("s")
    base = s * RPT

    z16 = jnp.zeros((16,), jnp.float32)

    def _zrow(i, _):
        z_v[i, pl.ds(0, 16)] = z16
        z_v[i, pl.ds(16, 16)] = z16
        return 0

    lax.fori_loop(0, ZR, _zrow, 0)

    def _zcp(j, _):
        pltpu.sync_copy(z_v, acc.at[pl.ds(base + j * ZR, ZR)])
        return 0

    lax.fori_loop(0, RPT // ZR, _zcp, 0)
    plsc.subcore_barrier()

    def _run(y):
        ngrp = NCH // NBUF

        def _issue_idx(g, p):
            j0 = s * NCH + g * NBUF
            for b in range(NBUF):
                slot = p * NBUF + b
                rid_v[pl.ds(slot * 16, 16)] = (
                    jnp.full((16,), 2 * (j0 + b), jnp.int32)
                    + jnp.minimum(lax.iota(jnp.int32, 16), 1))
                ob = rid_v.at[pl.ds(slot * 16, 2)]
                pltpu.async_copy(edges.at[ob], sd_v.at[pl.ds(2 * slot, 2)],
                                 sem_i)

        _issue_idx(0, 0)

        def _group(g, _):
            p = lax.rem(g, 2)

            @pl.when(g > 0)
            def _():
                for b in range(NBUF):
                    slot = (1 - p) * NBUF + b
                    pltpu.make_async_copy(
                        row_v.at[b], acc.at[sd_v.at[2 * slot + 1]],
                        sem_s).wait()

            dg = []
            for b in range(NBUF):
                slot = p * NBUF + b
                ob = rid_v.at[pl.ds(slot * 16, 2)]
                pltpu.make_async_copy(
                    edges.at[ob], sd_v.at[pl.ds(2 * slot, 2)], sem_i).wait()
                dg.append(pltpu.async_copy(
                    y.at[sd_v.at[2 * slot]], row_v.at[b], sem_g))

            @pl.when(g + 1 < ngrp)
            def _():
                _issue_idx(g + 1, 1 - p)

            for b in range(NBUF):
                slot = p * NBUF + b
                dg[b].wait()
                pltpu.async_copy(row_v.at[b], acc.at[sd_v.at[2 * slot + 1]],
                                 sem_s, add=True)
            return 0

        lax.fori_loop(0, ngrp, _group, 0)

        for b in range(NBUF):
            slot = ((ngrp - 1) % 2) * NBUF + b
            pltpu.make_async_copy(
                row_v.at[b], acc.at[sd_v.at[2 * slot + 1]], sem_s).wait()

    @pl.when(c == 0)
    def _():
        _run(y0)

    @pl.when(c == 1)
    def _():
        _run(y1)

    plsc.subcore_barrier()

    sl = pl.ds(base, RPT)

    @pl.when(c == 0)
    def _():
        pltpu.sync_copy(acc.at[sl], out0.at[sl])

    @pl.when(c == 1)
    def _():
        pltpu.sync_copy(acc.at[sl], out1.at[sl])


def _proj0_body(x_ref, w_ref, y0_ref, y1_ref):
    y = jnp.dot(x_ref[...], w_ref[...], preferred_element_type=jnp.float32)
    y0_ref[...] = y[:, :HALF]
    y1_ref[...] = y[:, HALF:]


def _proj0(x, w1):
    return pl.pallas_call(
        _proj0_body,
        grid=(NGRID,),
        in_specs=[
            pl.BlockSpec((NBLK, 128), lambda i: (i, 0)),
            pl.BlockSpec((128, HH), lambda i: (0, 0)),
        ],
        out_specs=[
            pl.BlockSpec((NBLK, HALF), lambda i: (i, 0)),
            pl.BlockSpec((NBLK, HALF), lambda i: (i, 0)),
        ],
        out_shape=[jax.ShapeDtypeStruct((NN, HALF), jnp.float32)] * 2,
    )(x, w1)


def _in_sp(shape):
    return pl.BlockSpec(shape, lambda p, i: (jnp.where(p == 0, i, 0), 0))


def _w_sp(shape):
    return pl.BlockSpec(shape, lambda p, i: (0, 0))


def _mlpbn_common(p, i, y0_ref, y1_ref, a0_ref, a1_ref, b1_ref, w2_ref,
                  b2_ref, g_ref, be_ref, v_s, st_s):
    @pl.when(p == 0)
    def _():
        z0 = y0_ref[...] + a0_ref[...]
        z1 = y1_ref[...] + a1_ref[...]
        z = jnp.concatenate([z0, z1], axis=1) + b1_ref[...]
        u = jnp.maximum(z, 0.0)
        v = jnp.dot(u, w2_ref[...],
                    preferred_element_type=jnp.float32) + b2_ref[...]
        v_s[pl.ds(i * NBLK, NBLK), :] = v

        @pl.when(i == 0)
        def _():
            st_s[...] = jnp.zeros_like(st_s)

        st_s[0:1, :] += jnp.sum(v, axis=0, keepdims=True)
        st_s[1:2, :] += jnp.sum(v * v, axis=0, keepdims=True)

    mean = st_s[0:1, :] * (1.0 / NN)
    var = st_s[1:2, :] * (1.0 / NN) - mean * mean
    inv = lax.rsqrt(var + 1e-5) * g_ref[...]
    vb = v_s[pl.ds(i * NBLK, NBLK), :]
    return jnp.maximum((vb - mean) * inv + be_ref[...], 0.0)


def _mlpbnproj_body(y0_ref, y1_ref, a0_ref, a1_ref, b1_ref, w2_ref, b2_ref,
                    g_ref, be_ref, w1n_ref, y0o_ref, y1o_ref, v_s, st_s):
    p = pl.program_id(0)
    i = pl.program_id(1)
    h = _mlpbn_common(p, i, y0_ref, y1_ref, a0_ref, a1_ref, b1_ref, w2_ref,
                      b2_ref, g_ref, be_ref, v_s, st_s)

    @pl.when(p == 1)
    def _():
        y = jnp.dot(h, w1n_ref[...], preferred_element_type=jnp.float32)
        y0o_ref[...] = y[:, :HALF]
        y1o_ref[...] = y[:, HALF:]


def _mlpbnproj(y0, y1, a0, a1, b1, w2, b2, g, be, w1n):
    return pl.pallas_call(
        _mlpbnproj_body,
        grid=(2, NGRID),
        in_specs=[
            _in_sp((NBLK, HALF)), _in_sp((NBLK, HALF)),
            _in_sp((NBLK, HALF)), _in_sp((NBLK, HALF)),
            _w_sp((1, HH)), _w_sp((HH, HH)), _w_sp((1, HH)),
            _w_sp((1, HH)), _w_sp((1, HH)), _w_sp((HH, HH)),
        ],
        out_specs=[
            pl.BlockSpec((NBLK, HALF), lambda p, i: (jnp.where(p == 1, i, 0), 0)),
            pl.BlockSpec((NBLK, HALF), lambda p, i: (jnp.where(p == 1, i, 0), 0)),
        ],
        out_shape=[jax.ShapeDtypeStruct((NN, HALF), jnp.float32)] * 2,
        scratch_shapes=[
            pltpu.VMEM((NN, HH), jnp.float32),
            pltpu.VMEM((2, HH), jnp.float32),
        ],
    )(y0, y1, a0, a1, b1, w2, b2, g, be, w1n)


def _mlpbnpool_body(y0_ref, y1_ref, a0_ref, a1_ref, b1_ref, w2_ref, b2_ref,
                    g_ref, be_ref, batch_ref, pooled_ref, cnt_ref, v_s, st_s):
    p = pl.program_id(0)
    i = pl.program_id(1)
    h = _mlpbn_common(p, i, y0_ref, y1_ref, a0_ref, a1_ref, b1_ref, w2_ref,
                      b2_ref, g_ref, be_ref, v_s, st_s)

    @pl.when(p == 1)
    def _():
        b = batch_ref[0, 0, :]
        gid = lax.broadcasted_iota(jnp.int32, (GG, NBLK), 0)
        onehot = jnp.where(gid == b[None, :], 1.0, 0.0)

        @pl.when(i == 0)
        def _():
            pooled_ref[...] = jnp.zeros_like(pooled_ref)
            cnt_ref[...] = jnp.zeros_like(cnt_ref)

        pooled_ref[...] += jnp.dot(onehot, h,
                                   preferred_element_type=jnp.float32)
        cnt_ref[...] += jnp.sum(onehot, axis=1, keepdims=True)


def _mlpbnpool(y0, y1, a0, a1, b1, w2, b2, g, be, batch3):
    return pl.pallas_call(
        _mlpbnpool_body,
        grid=(2, NGRID),
        in_specs=[
            _in_sp((NBLK, HALF)), _in_sp((NBLK, HALF)),
            _in_sp((NBLK, HALF)), _in_sp((NBLK, HALF)),
            _w_sp((1, HH)), _w_sp((HH, HH)), _w_sp((1, HH)),
            _w_sp((1, HH)), _w_sp((1, HH)),
            pl.BlockSpec((1, 1, NBLK),
                         lambda p, i: (jnp.where(p == 1, i, 0), 0, 0)),
        ],
        out_specs=[
            pl.BlockSpec((GG, HH), lambda p, i: (0, 0)),
            pl.BlockSpec((GG, 1), lambda p, i: (0, 0)),
        ],
        out_shape=[
            jax.ShapeDtypeStruct((GG, HH), jnp.float32),
            jax.ShapeDtypeStruct((GG, 1), jnp.float32),
        ],
        scratch_shapes=[
            pltpu.VMEM((NN, HH), jnp.float32),
            pltpu.VMEM((2, HH), jnp.float32),
        ],
    )(y0, y1, a0, a1, b1, w2, b2, g, be, batch3)


def _final_body(pooled_ref, cnt_ref, wp_ref, bp_ref, o_ref):
    pm = pooled_ref[...] / jnp.maximum(cnt_ref[...], 1.0)
    o = jnp.dot(pm, wp_ref[...], preferred_element_type=jnp.float32) + bp_ref[...]
    nrm = jnp.sqrt(jnp.sum(o * o, axis=1, keepdims=True))
    o_ref[...] = o / jnp.maximum(nrm, 1e-12)


def _final(pooled, cnt, wp, bp):
    return pl.pallas_call(
        _final_body,
        out_shape=jax.ShapeDtypeStruct((GG, HH), jnp.float32),
    )(pooled, cnt, wp, bp)


def kernel(x, edge_index, batch,
           W1_0, b1_0, W2_0, b2_0, g_0, be_0,
           W1_1, b1_1, W2_1, b2_1, g_1, be_1,
           W1_2, b1_2, W2_2, b2_2, g_2, be_2,
           Wp, bp):
    pad = EP - EE
    srcs = jnp.concatenate(
        [edge_index[0], jnp.zeros((pad,), jnp.int32)]).reshape(NTILE * NCH, ECH)
    dsts = jnp.concatenate(
        [edge_index[1], jnp.full((pad,), NP - 1, jnp.int32)]).reshape(NTILE * NCH, ECH)
    edges = jnp.stack([srcs, dsts], axis=1).reshape(2 * NTILE * NCH, ECH)
    batch3 = batch.reshape(NGRID, 1, NBLK)
    r = lambda a: a.reshape(1, HH)

    layers = [
        (b1_0, W2_0, b2_0, g_0, be_0, W1_1),
        (b1_1, W2_1, b2_1, g_1, be_1, W1_2),
        (b1_2, W2_2, b2_2, g_2, be_2, None),
    ]

    y0, y1 = _proj0(x, W1_0)
    for b1, w2, b2, g, be, w1n in layers:
        a0, a1 = _make_sc_agg()(y0, y1, edges)
        if w1n is not None:
            y0, y1 = _mlpbnproj(y0, y1, a0, a1, r(b1), w2, r(b2),
                                r(g), r(be), w1n)
        else:
            pooled, cnt = _mlpbnpool(y0, y1, a0, a1, r(b1), w2, r(b2),
                                     r(g), r(be), batch3)
    return _final(pooled, cnt, Wp, r(bp))


# speedup vs baseline: 1.0034x; 1.0034x over previous
import functools

import jax
import jax.numpy as jnp
from jax import lax
from jax.experimental import pallas as pl
from jax.experimental.pallas import tpu as pltpu
from jax.experimental.pallas import tpu_sc as plsc

NN = 50000
EE = 800000
GG = 512
HH = 64
HALF = 32
NBLK = 2000
NGRID = NN // NBLK
NTILE = 16
ECH = 128
NCH = 392
EP = NTILE * NCH * ECH
NP = 50048
RPT = NP // NTILE
ZR = 136
NBUF = 4


@functools.cache
def _make_sc_agg():
    mesh = plsc.VectorSubcoreMesh(
        core_axis_name="c", subcore_axis_name="s", num_cores=2, num_subcores=16)
    return functools.partial(
        pl.kernel,
        out_type=(
            jax.ShapeDtypeStruct((NP, HALF), jnp.float32),
            jax.ShapeDtypeStruct((NP, HALF), jnp.float32),
        ),
        mesh=mesh,
        scratch_types=[
            pltpu.VMEM((4 * NBUF, ECH), jnp.int32),
            pltpu.VMEM((2 * NBUF * 16,), jnp.int32),
            pltpu.VMEM((NBUF, ECH, HALF), jnp.float32),
            pltpu.VMEM((ZR, HALF), jnp.float32),
            pltpu.VMEM_SHARED((NP, HALF), jnp.float32),
            pltpu.SemaphoreType.DMA,
            pltpu.SemaphoreType.DMA,
            pltpu.SemaphoreType.DMA,
        ],
        compiler_params=pltpu.CompilerParams(use_tc_tiling_on_sc=False),
    )(_sc_agg_body)


def _sc_agg_body(y0, y1, edges, out0, out1,
                 sd_v, rid_v, row_v, z_v, acc, sem_i, sem_g, sem_s):
    c = lax.axis_index("c")
    s = lax.axis_index("s")
    base = s * RPT

    z16 = jnp.zeros((16,), jnp.float32)

    def _zrow(i, _):
        z_v[i, pl.ds(0, 16)] = z16
        z_v[i, pl.ds(16, 16)] = z16
        return 0

    lax.fori_loop(0, ZR, _zrow, 0)

    def _zcp(j, _):
        pltpu.sync_copy(z_v, acc.at[pl.ds(base + j * ZR, ZR)])
        return 0

    lax.fori_loop(0, RPT // ZR, _zcp, 0)
    plsc.subcore_barrier()

    def _run(y):
        ngrp = NCH // NBUF

        def _issue_idx(g, p):
            j0 = s * NCH + g * NBUF
            for b in range(NBUF):
                slot = p * NBUF + b
                rid_v[pl.ds(slot * 16, 16)] = (
                    jnp.full((16,), 2 * (j0 + b), jnp.int32)
                    + jnp.minimum(lax.iota(jnp.int32, 16), 1))
                ob = rid_v.at[pl.ds(slot * 16, 2)]
                pltpu.async_copy(edges.at[ob], sd_v.at[pl.ds(2 * slot, 2)],
                                 sem_i)

        _issue_idx(0, 0)

        def _group(g, _):
            p = lax.rem(g, 2)

            @pl.when(g > 0)
            def _():
                for b in range(NBUF):
                    slot = (1 - p) * NBUF + b
                    pltpu.make_async_copy(
                        row_v.at[b], acc.at[sd_v.at[2 * slot + 1]],
                        sem_s).wait()

            dg = []
            for b in range(NBUF):
                slot = p * NBUF + b
                ob = rid_v.at[pl.ds(slot * 16, 2)]
                pltpu.make_async_copy(
                    edges.at[ob], sd_v.at[pl.ds(2 * slot, 2)], sem_i).wait()
                dg.append(pltpu.async_copy(
                    y.at[sd_v.at[2 * slot]], row_v.at[b], sem_g))

            @pl.when(g + 1 < ngrp)
            def _():
                _issue_idx(g + 1, 1 - p)

            for b in range(NBUF):
                slot = p * NBUF + b
                dg[b].wait()
                pltpu.async_copy(row_v.at[b], acc.at[sd_v.at[2 * slot + 1]],
                                 sem_s, add=True)
            return 0

        lax.fori_loop(0, ngrp, _group, 0)

        for b in range(NBUF):
            slot = ((ngrp - 1) % 2) * NBUF + b
            pltpu.make_async_copy(
                row_v.at[b], acc.at[sd_v.at[2 * slot + 1]], sem_s).wait()

    @pl.when(c == 0)
    def _():
        _run(y0)

    @pl.when(c == 1)
    def _():
        _run(y1)

    plsc.subcore_barrier()

    sl = pl.ds(base, RPT)

    @pl.when(c == 0)
    def _():
        pltpu.sync_copy(acc.at[sl], out0.at[sl])

    @pl.when(c == 1)
    def _():
        pltpu.sync_copy(acc.at[sl], out1.at[sl])


def _proj0_body(x_ref, w_ref, y0_ref, y1_ref):
    y = jnp.dot(x_ref[...], w_ref[...], preferred_element_type=jnp.float32)
    y0_ref[...] = y[:, :HALF]
    y1_ref[...] = y[:, HALF:]


def _proj0(x, w1):
    return pl.pallas_call(
        _proj0_body,
        grid=(NGRID,),
        in_specs=[
            pl.BlockSpec((NBLK, 128), lambda i: (i, 0)),
            pl.BlockSpec((128, HH), lambda i: (0, 0)),
        ],
        out_specs=[
            pl.BlockSpec((NBLK, HALF), lambda i: (i, 0)),
            pl.BlockSpec((NBLK, HALF), lambda i: (i, 0)),
        ],
        out_shape=[jax.ShapeDtypeStruct((NP, HALF), jnp.float32)] * 2,
    )(x, w1)


def _in_sp(shape):
    return pl.BlockSpec(shape, lambda p, i: (jnp.where(p == 0, i, 0), 0))


def _w_sp(shape):
    return pl.BlockSpec(shape, lambda p, i: (0, 0))


def _mlpbn_common(p, i, y0_ref, y1_ref, a0_ref, a1_ref, b1_ref, w2_ref,
                  b2_ref, g_ref, be_ref, v_s, st_s):
    @pl.when(p == 0)
    def _():
        z0 = y0_ref[...] + a0_ref[...]
        z1 = y1_ref[...] + a1_ref[...]
        z = jnp.concatenate([z0, z1], axis=1) + b1_ref[...]
        u = jnp.maximum(z, 0.0)
        v = jnp.dot(u, w2_ref[...],
                    preferred_element_type=jnp.float32) + b2_ref[...]
        v_s[pl.ds(i * NBLK, NBLK), :] = v

        @pl.when(i == 0)
        def _():
            st_s[...] = jnp.zeros_like(st_s)

        st_s[0:1, :] += jnp.sum(v, axis=0, keepdims=True)
        st_s[1:2, :] += jnp.sum(v * v, axis=0, keepdims=True)

    mean = st_s[0:1, :] * (1.0 / NN)
    var = st_s[1:2, :] * (1.0 / NN) - mean * mean
    inv = lax.rsqrt(var + 1e-5) * g_ref[...]
    vb = v_s[pl.ds(i * NBLK, NBLK), :]
    return jnp.maximum((vb - mean) * inv + be_ref[...], 0.0)


def _mlpbnproj_body(y0_ref, y1_ref, a0_ref, a1_ref, b1_ref, w2_ref, b2_ref,
                    g_ref, be_ref, w1n_ref, y0o_ref, y1o_ref, v_s, st_s):
    p = pl.program_id(0)
    i = pl.program_id(1)
    h = _mlpbn_common(p, i, y0_ref, y1_ref, a0_ref, a1_ref, b1_ref, w2_ref,
                      b2_ref, g_ref, be_ref, v_s, st_s)

    @pl.when(p == 1)
    def _():
        y = jnp.dot(h, w1n_ref[...], preferred_element_type=jnp.float32)
        y0o_ref[...] = y[:, :HALF]
        y1o_ref[...] = y[:, HALF:]


def _mlpbnproj(y0, y1, a0, a1, b1, w2, b2, g, be, w1n):
    return pl.pallas_call(
        _mlpbnproj_body,
        grid=(2, NGRID),
        in_specs=[
            _in_sp((NBLK, HALF)), _in_sp((NBLK, HALF)),
            _in_sp((NBLK, HALF)), _in_sp((NBLK, HALF)),
            _w_sp((1, HH)), _w_sp((HH, HH)), _w_sp((1, HH)),
            _w_sp((1, HH)), _w_sp((1, HH)), _w_sp((HH, HH)),
        ],
        out_specs=[
            pl.BlockSpec((NBLK, HALF), lambda p, i: (jnp.where(p == 1, i, 0), 0)),
            pl.BlockSpec((NBLK, HALF), lambda p, i: (jnp.where(p == 1, i, 0), 0)),
        ],
        out_shape=[jax.ShapeDtypeStruct((NP, HALF), jnp.float32)] * 2,
        scratch_shapes=[
            pltpu.VMEM((NN, HH), jnp.float32),
            pltpu.VMEM((2, HH), jnp.float32),
        ],
    )(y0, y1, a0, a1, b1, w2, b2, g, be, w1n)


def _mlpbnpool_body(y0_ref, y1_ref, a0_ref, a1_ref, b1_ref, w2_ref, b2_ref,
                    g_ref, be_ref, batch_ref, pooled_ref, cnt_ref, v_s, st_s):
    p = pl.program_id(0)
    i = pl.program_id(1)
    h = _mlpbn_common(p, i, y0_ref, y1_ref, a0_ref, a1_ref, b1_ref, w2_ref,
                      b2_ref, g_ref, be_ref, v_s, st_s)

    @pl.when(p == 1)
    def _():
        b = batch_ref[0, 0, :]
        gid = lax.broadcasted_iota(jnp.int32, (GG, NBLK), 0)
        onehot = jnp.where(gid == b[None, :], 1.0, 0.0)

        @pl.when(i == 0)
        def _():
            pooled_ref[...] = jnp.zeros_like(pooled_ref)
            cnt_ref[...] = jnp.zeros_like(cnt_ref)

        pooled_ref[...] += jnp.dot(onehot, h,
                                   preferred_element_type=jnp.float32)
        cnt_ref[...] += jnp.sum(onehot, axis=1, keepdims=True)


def _mlpbnpool(y0, y1, a0, a1, b1, w2, b2, g, be, batch3):
    return pl.pallas_call(
        _mlpbnpool_body,
        grid=(2, NGRID),
        in_specs=[
            _in_sp((NBLK, HALF)), _in_sp((NBLK, HALF)),
            _in_sp((NBLK, HALF)), _in_sp((NBLK, HALF)),
            _w_sp((1, HH)), _w_sp((HH, HH)), _w_sp((1, HH)),
            _w_sp((1, HH)), _w_sp((1, HH)),
            pl.BlockSpec((1, 1, NBLK),
                         lambda p, i: (jnp.where(p == 1, i, 0), 0, 0)),
        ],
        out_specs=[
            pl.BlockSpec((GG, HH), lambda p, i: (0, 0)),
            pl.BlockSpec((GG, 1), lambda p, i: (0, 0)),
        ],
        out_shape=[
            jax.ShapeDtypeStruct((GG, HH), jnp.float32),
            jax.ShapeDtypeStruct((GG, 1), jnp.float32),
        ],
        scratch_shapes=[
            pltpu.VMEM((NN, HH), jnp.float32),
            pltpu.VMEM((2, HH), jnp.float32),
        ],
    )(y0, y1, a0, a1, b1, w2, b2, g, be, batch3)


def _final_body(pooled_ref, cnt_ref, wp_ref, bp_ref, o_ref):
    pm = pooled_ref[...] / jnp.maximum(cnt_ref[...], 1.0)
    o = jnp.dot(pm, wp_ref[...], preferred_element_type=jnp.float32) + bp_ref[...]
    nrm = jnp.sqrt(jnp.sum(o * o, axis=1, keepdims=True))
    o_ref[...] = o / jnp.maximum(nrm, 1e-12)


def _final(pooled, cnt, wp, bp):
    return pl.pallas_call(
        _final_body,
        out_shape=jax.ShapeDtypeStruct((GG, HH), jnp.float32),
    )(pooled, cnt, wp, bp)


def kernel(x, edge_index, batch,
           W1_0, b1_0, W2_0, b2_0, g_0, be_0,
           W1_1, b1_1, W2_1, b2_1, g_1, be_1,
           W1_2, b1_2, W2_2, b2_2, g_2, be_2,
           Wp, bp):
    pad = EP - EE
    srcs = jnp.concatenate(
        [edge_index[0], jnp.zeros((pad,), jnp.int32)]).reshape(NTILE * NCH, ECH)
    dsts = jnp.concatenate(
        [edge_index[1], jnp.full((pad,), NP - 1, jnp.int32)]).reshape(NTILE * NCH, ECH)
    edges = jnp.stack([srcs, dsts], axis=1).reshape(2 * NTILE * NCH, ECH)
    batch3 = batch.reshape(NGRID, 1, NBLK)
    r = lambda a: a.reshape(1, HH)

    layers = [
        (b1_0, W2_0, b2_0, g_0, be_0, W1_1),
        (b1_1, W2_1, b2_1, g_1, be_1, W1_2),
        (b1_2, W2_2, b2_2, g_2, be_2, None),
    ]

    y0, y1 = _proj0(x, W1_0)
    for b1, w2, b2, g, be, w1n in layers:
        a0, a1 = _make_sc_agg()(y0, y1, edges)
        if w1n is not None:
            y0, y1 = _mlpbnproj(y0, y1, a0, a1, r(b1), w2, r(b2),
                                r(g), r(be), w1n)
        else:
            pooled, cnt = _mlpbnpool(y0, y1, a0, a1, r(b1), w2, r(b2),
                                     r(g), r(be), batch3)
    return _final(pooled, cnt, Wp, r(bp))
